# baseline (speedup 1.0000x reference)
import functools

import jax
import jax.numpy as jnp
from jax import lax
from jax.experimental import pallas as pl
from jax.experimental.pallas import tpu as pltpu
from jax.experimental.pallas import tpu_sc as plsc

NUM_CLASSES = 1024
EMBED_DIM = 1024
BATCH = 16384

B_SC = 10240
B_TC = BATCH - B_SC

NC = 2
NS = 16
L = 16
NW = NC * NS
ROWS_PER_W = B_SC // NW
GROUP = 16
NGROUPS = ROWS_PER_W // GROUP
NCHUNK = EMBED_DIM // L
NBUF = 4

TB = 1024
NBLK = B_TC // TB
NBLK0 = B_SC // TB


def _rsqrt(s):
    i = lax.bitcast_convert_type(s, jnp.int32)
    i = jnp.int32(0x5F3759DF) - lax.shift_right_logical(i, 1)
    r = lax.bitcast_convert_type(i, jnp.float32)
    half = s * 0.5
    for _ in range(3):
        r = r * (1.5 - half * r * r)
    return r


def _tile_body(emb_hbm, lab_hbm, anc_hbm, out_hbm,
               idx_v, emb_v, anc_v, anc_sh, dots_v, sqs_v, acc_v,
               sem_e, sem_a):
    sid = lax.axis_index("s")
    wid = sid * NC + lax.axis_index("c")
    base = wid * ROWS_PER_W

    for half in range(NUM_CLASSES // NS // GROUP):
        rows0 = sid * (NUM_CLASSES // NS) + half * GROUP
        pltpu.sync_copy(anc_hbm.at[pl.ds(rows0, GROUP)], emb_v.at[0])

        def pack_row(r, _):
            for t in range(NCHUNK // 2):
                lo = emb_v[0, r, pl.ds(t * 2 * L, L)]
                hi = emb_v[0, r, pl.ds(t * 2 * L + L, L)]
                packed = plsc.pack(lo, hi,
                                   format=plsc.PackFormat.INTERLEAVED)
                anc_v[0, r, pl.ds(t * L, L)] = plsc.bitcast(packed,
                                                            jnp.int32)
            return 0

        lax.fori_loop(0, GROUP, pack_row, 0)
        pltpu.sync_copy(anc_v.at[0], anc_sh.at[pl.ds(rows0, GROUP)])
    plsc.subcore_barrier()

    pltpu.sync_copy(lab_hbm.at[pl.ds(base, ROWS_PER_W)], idx_v)

    acc_v[...] = jnp.zeros((L,), jnp.float32)

    def start_e(g, b):
        pltpu.async_copy(
            emb_hbm.at[pl.ds(base + g * GROUP, GROUP)], emb_v.at[b], sem_e[b])

    def start_a(g, b):
        labs = idx_v[pl.ds(g * GROUP, GROUP)]
        for j in range(GROUP):
            pltpu.async_copy(
                anc_sh.at[pl.ds(labs[j], 1)],
                anc_v.at[b, pl.ds(j, 1)], sem_a[b])

    def wait(b, ba):
        pltpu.make_async_copy(
            emb_hbm.at[pl.ds(0, GROUP)], emb_v.at[b], sem_e[b]).wait()
        pltpu.make_async_copy(
            anc_sh.at[pl.ds(0, GROUP)], anc_v.at[ba], sem_a[ba]).wait()

    def compute(b, ba):
        def row_body(r, _):
            zero = jnp.zeros((L,), jnp.float32)
            zbf = jnp.zeros((2 * L,), jnp.bfloat16)
            df = [zero, zero]
            sf = [zero, zero]
            for w in range(4):
                accd = zbf
                accs = zbf
                for u in range(8):
                    t = w * 8 + u
                    ab = plsc.bitcast(anc_v[ba, r, pl.ds(t * L, L)],
                                      jnp.bfloat16)
                    e0 = emb_v[b, r, pl.ds((2 * t) * L, L)]
                    e1 = emb_v[b, r, pl.ds((2 * t + 1) * L, L)]
                    e01 = plsc.pack(e0, e1,
                                    format=plsc.PackFormat.INTERLEAVED)
                    accd = accd + e01 * ab
                    accs = accs + e01 * e01
                dl, dh = plsc.unpack(accd,
                                     format=plsc.PackFormat.INTERLEAVED)
                sl, sh = plsc.unpack(accs,
                                     format=plsc.PackFormat.INTERLEAVED)
                df[0] = df[0] + dl
                df[1] = df[1] + dh
                sf[0] = sf[0] + sl
                sf[1] = sf[1] + sh
            dots_v[pl.ds(r * L, L)] = df[0] + df[1]
            sqs_v[pl.ds(r * L, L)] = sf[0] + sf[1]
            return 0

        lax.fori_loop(0, GROUP, row_body, 0)

        rows = lax.iota(jnp.int32, L) * L
        tot_d = jnp.zeros((L,), jnp.float32)
        tot_s = jnp.zeros((L,), jnp.float32)
        for j in range(L):
            idx = rows + j
            tot_d = tot_d + plsc.load_gather(dots_v, [idx])
            tot_s = tot_s + plsc.load_gather(sqs_v, [idx])

        acc_v[...] = acc_v[...] + tot_d * _rsqrt(tot_s)

    for p in range(NBUF - 1):
        start_e(p, p)
    for p in range(2):
        start_a(p, p)

    @pl.loop(0, NGROUPS, step=NBUF)
    def _(g):
        for b in range(NBUF):
            gg = g + b
            nxt = gg + NBUF - 1
            ba = b % 2

            @pl.when(nxt < NGROUPS)
            def _():
                start_e(nxt, (b + NBUF - 1) % NBUF)

            wait(b, ba)
            compute(b, ba)

            nxta = gg + 2

            @pl.when(nxta < NGROUPS)
            def _():
                start_a(nxta, ba)

    pltpu.sync_copy(acc_v, out_hbm.at[wid])


@jax.jit
def _oar_partials(embeddings, labels, anchors):
    mesh = plsc.VectorSubcoreMesh(
        core_axis_name="c", subcore_axis_name="s", num_cores=NC)
    k = pl.kernel(
        _tile_body,
        out_type=jax.ShapeDtypeStruct((NW, L), jnp.float32),
        mesh=mesh,
        compiler_params=pltpu.CompilerParams(needs_layout_passes=False),
        scratch_types=[
            pltpu.VMEM((ROWS_PER_W,), jnp.int32),
            pltpu.VMEM((NBUF, GROUP, EMBED_DIM), jnp.float32),
            pltpu.VMEM((2, GROUP, EMBED_DIM // 2), jnp.int32),
            pltpu.VMEM_SHARED((NUM_CLASSES, EMBED_DIM // 2), jnp.int32),
            pltpu.VMEM((L * L,), jnp.float32),
            pltpu.VMEM((L * L,), jnp.float32),
            pltpu.VMEM((L,), jnp.float32),
            [pltpu.SemaphoreType.DMA] * NBUF,
            [pltpu.SemaphoreType.DMA] * 2,
        ],
    )
    return k(embeddings, labels, anchors)


def _tc_body(emb_ref, lab_ref, anc_ref, out_ref, anc_bf_ref):
    @pl.when(pl.program_id(0) == 0)
    def _():
        anc_bf_ref[...] = anc_ref[...].astype(jnp.bfloat16)

    e = emb_ref[...]
    lab = lab_ref[0]
    onehot_t = (lab == lax.broadcasted_iota(
        jnp.int32, (NUM_CLASSES, TB), 0)).astype(jnp.bfloat16)
    g = lax.dot_general(onehot_t, anc_bf_ref[...],
                        dimension_numbers=(((0,), (0,)), ((), ())),
                        preferred_element_type=jnp.float32)
    dots = jnp.sum(e * g, axis=1, keepdims=True)
    sq = jnp.sum(e * e, axis=1, keepdims=True)
    norm = jnp.maximum(jnp.sqrt(sq), 1e-12)
    out_ref[0] = dots / norm


def _tc_partials(embeddings, labels, anchors):
    lab3 = labels.reshape(BATCH // TB, 1, TB)
    return pl.pallas_call(
        _tc_body,
        grid=(NBLK,),
        in_specs=[
            pl.BlockSpec((TB, EMBED_DIM), lambda i: (i + NBLK0, 0)),
            pl.BlockSpec((1, 1, TB), lambda i: (i + NBLK0, 0, 0)),
            pl.BlockSpec((NUM_CLASSES, NUM_CLASSES), lambda i: (0, 0)),
        ],
        out_specs=pl.BlockSpec((1, TB, 1), lambda i: (i, 0, 0)),
        out_shape=jax.ShapeDtypeStruct((NBLK, TB, 1), jnp.float32),
        scratch_shapes=[
            pltpu.VMEM((NUM_CLASSES, NUM_CLASSES), jnp.bfloat16)],
    )(embeddings, lab3, anchors)


def kernel(embeddings, labels, anchors):
    sc = _oar_partials(embeddings, labels, anchors)
    tc = _tc_partials(embeddings, labels, anchors)
    total = jnp.sum(sc) + jnp.sum(tc)
    return (1.0 - total / BATCH).astype(jnp.float32)


# speedup vs baseline: 1.2552x; 1.2552x over previous
import functools

import jax
import jax.numpy as jnp
from jax import lax
from jax.experimental import pallas as pl
from jax.experimental.pallas import tpu as pltpu
from jax.experimental.pallas import tpu_sc as plsc

NUM_CLASSES = 1024
EMBED_DIM = 1024
BATCH = 16384

B_SC = 10240
B_TC = BATCH - B_SC

NC = 2
NS = 16
L = 16
NW = NC * NS
ROWS_PER_W = B_SC // NW
GROUP = 16
NGROUPS = ROWS_PER_W // GROUP
NCHUNK = EMBED_DIM // L
NBUF = 4

TB = 1024
NBLK = B_TC // TB
NBLK0 = B_SC // TB


def _rsqrt(s):
    i = lax.bitcast_convert_type(s, jnp.int32)
    i = jnp.int32(0x5F3759DF) - lax.shift_right_logical(i, 1)
    r = lax.bitcast_convert_type(i, jnp.float32)
    half = s * 0.5
    for _ in range(3):
        r = r * (1.5 - half * r * r)
    return r


def _tile_body(emb_hbm, lab_hbm, anc_hbm, out_hbm,
               idx_v, emb_v, anc_v, dots_v, sqs_v, acc_v,
               sem_e, sem_a):
    sid = lax.axis_index("s")
    wid = sid * NC + lax.axis_index("c")
    base = wid * ROWS_PER_W

    pltpu.sync_copy(lab_hbm.at[pl.ds(base, ROWS_PER_W)], idx_v)

    acc_v[...] = jnp.zeros((L,), jnp.float32)

    def start(g, b):
        pltpu.async_copy(
            emb_hbm.at[pl.ds(base + g * GROUP, GROUP)], emb_v.at[b], sem_e[b])
        idxs = idx_v[pl.ds(g * GROUP, GROUP)]
        pltpu.async_copy(anc_hbm.at[idxs], anc_v.at[b], sem_a[b])

    def wait(b):
        pltpu.make_async_copy(
            emb_hbm.at[pl.ds(0, GROUP)], emb_v.at[b], sem_e[b]).wait()
        pltpu.make_async_copy(
            anc_hbm.at[pl.ds(0, GROUP)], anc_v.at[b], sem_a[b]).wait()

    def compute(b):
        def row_body(r, _):
            zero = jnp.zeros((L,), jnp.float32)
            zbf = jnp.zeros((2 * L,), jnp.bfloat16)
            df = [zero, zero]
            sf = [zero, zero]
            for w in range(4):
                accd = zbf
                accs = zbf
                for u in range(8):
                    t = w * 8 + u
                    ab = plsc.bitcast(anc_v[b, r, pl.ds(t * L, L)],
                                      jnp.bfloat16)
                    e0 = emb_v[b, r, pl.ds((2 * t) * L, L)]
                    e1 = emb_v[b, r, pl.ds((2 * t + 1) * L, L)]
                    e01 = plsc.pack(e0, e1,
                                    format=plsc.PackFormat.INTERLEAVED)
                    accd = accd + e01 * ab
                    accs = accs + e01 * e01
                dl, dh = plsc.unpack(accd,
                                     format=plsc.PackFormat.INTERLEAVED)
                sl, sh = plsc.unpack(accs,
                                     format=plsc.PackFormat.INTERLEAVED)
                df[0] = df[0] + dl
                df[1] = df[1] + dh
                sf[0] = sf[0] + sl
                sf[1] = sf[1] + sh
            dots_v[pl.ds(r * L, L)] = df[0] + df[1]
            sqs_v[pl.ds(r * L, L)] = sf[0] + sf[1]
            return 0

        lax.fori_loop(0, GROUP, row_body, 0)

        rows = lax.iota(jnp.int32, L) * L
        tot_d = jnp.zeros((L,), jnp.float32)
        tot_s = jnp.zeros((L,), jnp.float32)
        for j in range(L):
            idx = rows + j
            tot_d = tot_d + plsc.load_gather(dots_v, [idx])
            tot_s = tot_s + plsc.load_gather(sqs_v, [idx])

        acc_v[...] = acc_v[...] + tot_d * _rsqrt(tot_s)

    for p in range(NBUF - 1):
        start(p, p)

    @pl.loop(0, NGROUPS, step=NBUF)
    def _(g):
        for b in range(NBUF):
            gg = g + b
            nxt = gg + NBUF - 1

            @pl.when(nxt < NGROUPS)
            def _():
                start(nxt, (b + NBUF - 1) % NBUF)

            wait(b)
            compute(b)

    pltpu.sync_copy(acc_v, out_hbm.at[wid])


@jax.jit
def _oar_partials(embeddings, labels, anchors):
    anchors = (anchors.astype(jnp.bfloat16)
               .reshape(NUM_CLASSES, EMBED_DIM // 32, 2, L)
               .transpose(0, 1, 3, 2)
               .reshape(NUM_CLASSES, EMBED_DIM // 2, 2))
    anchors = lax.bitcast_convert_type(anchors, jnp.int32)
    mesh = plsc.VectorSubcoreMesh(
        core_axis_name="c", subcore_axis_name="s", num_cores=NC)
    k = pl.kernel(
        _tile_body,
        out_type=jax.ShapeDtypeStruct((NW, L), jnp.float32),
        mesh=mesh,
        compiler_params=pltpu.CompilerParams(needs_layout_passes=False),
        scratch_types=[
            pltpu.VMEM((ROWS_PER_W,), jnp.int32),
            pltpu.VMEM((NBUF, GROUP, EMBED_DIM), jnp.float32),
            pltpu.VMEM((NBUF, GROUP, EMBED_DIM // 2), jnp.int32),
            pltpu.VMEM((L * L,), jnp.float32),
            pltpu.VMEM((L * L,), jnp.float32),
            pltpu.VMEM((L,), jnp.float32),
            [pltpu.SemaphoreType.DMA] * NBUF,
            [pltpu.SemaphoreType.DMA] * NBUF,
        ],
    )
    return k(embeddings, labels, anchors)


def _tc_body(emb_ref, lab_ref, anc_ref, out_ref, anc_bf_ref):
    @pl.when(pl.program_id(0) == 0)
    def _():
        anc_bf_ref[...] = anc_ref[...].astype(jnp.bfloat16)

    e = emb_ref[...]
    lab = lab_ref[0]
    onehot_t = (lab == lax.broadcasted_iota(
        jnp.int32, (NUM_CLASSES, TB), 0)).astype(jnp.bfloat16)
    g = lax.dot_general(onehot_t, anc_bf_ref[...],
                        dimension_numbers=(((0,), (0,)), ((), ())),
                        preferred_element_type=jnp.float32)
    dots = jnp.sum(e * g, axis=1, keepdims=True)
    sq = jnp.sum(e * e, axis=1, keepdims=True)
    norm = jnp.maximum(jnp.sqrt(sq), 1e-12)
    out_ref[0] = dots / norm


def _tc_partials(embeddings, labels, anchors):
    lab3 = labels.reshape(BATCH // TB, 1, TB)
    return pl.pallas_call(
        _tc_body,
        grid=(NBLK,),
        in_specs=[
            pl.BlockSpec((TB, EMBED_DIM), lambda i: (i + NBLK0, 0)),
            pl.BlockSpec((1, 1, TB), lambda i: (i + NBLK0, 0, 0)),
            pl.BlockSpec((NUM_CLASSES, NUM_CLASSES), lambda i: (0, 0)),
        ],
        out_specs=pl.BlockSpec((1, TB, 1), lambda i: (i, 0, 0)),
        out_shape=jax.ShapeDtypeStruct((NBLK, TB, 1), jnp.float32),
        scratch_shapes=[
            pltpu.VMEM((NUM_CLASSES, NUM_CLASSES), jnp.bfloat16)],
    )(embeddings, lab3, anchors)


def kernel(embeddings, labels, anchors):
    sc = _oar_partials(embeddings, labels, anchors)
    tc = _tc_partials(embeddings, labels, anchors)
    total = jnp.sum(sc) + jnp.sum(tc)
    return (1.0 - total / BATCH).astype(jnp.float32)
